# Initial kernel scaffold; baseline (speedup 1.0000x reference)
import jax
import jax.numpy as jnp
from jax.experimental import pallas as pl


def kernel(x, rw_pos_enc, edge_attr, edge_weight, node_weight, patch_pe, y_EA, params, edge_index, batch, subgraphs_nodes_mapper, combined_subgraphs, subgraphs_edges_mapper, subgraphs_batch, call_n_patches, context_subgraph_idx, target_subgraph_idxs):
    raise NotImplementedError("write your pallas kernel here")


# speedup vs baseline: 1.2290x; 1.2290x over previous
import functools

import jax
import jax.numpy as jnp
from jax import lax
from jax.experimental import pallas as pl
from jax.experimental.pallas import tpu as pltpu
from jax.experimental.pallas import tpu_sc as plsc

NC = 2
NS = 16
LN = 16
H = 128
HH = 64
F32 = jnp.float32


def _mesh():
    return plsc.VectorSubcoreMesh(
        core_axis_name="c", subcore_axis_name="s", num_cores=NC, num_subcores=NS)


@functools.lru_cache(maxsize=None)
def _sc_gather_fn(M, D):
    C = 80
    NW = NC * NS
    per_w = M // NW
    n_ch = per_w // C
    assert per_w % C == 0 and per_w * NW == M

    def body(table, idx, out, idx_v, rows_v, sem):
        wid = lax.axis_index("s") * NC + lax.axis_index("c")
        base = wid * per_w

        def chunk(j, carry):
            off = base + j * C
            pltpu.sync_copy(idx.at[pl.ds(off, C)], idx_v)
            pltpu.async_copy(table.at[idx_v], rows_v, sem).wait()
            pltpu.sync_copy(rows_v, out.at[pl.ds(off, C)])
            return carry

        lax.fori_loop(0, n_ch, chunk, 0)

    return pl.kernel(
        body,
        out_type=jax.ShapeDtypeStruct((M, D), F32),
        mesh=_mesh(),
        compiler_params=pltpu.CompilerParams(use_tc_tiling_on_sc=False, needs_layout_passes=False),
        scratch_types=[
            pltpu.VMEM((C,), jnp.int32),
            pltpu.VMEM((C, D), F32),
            pltpu.SemaphoreType.DMA,
        ])


@functools.lru_cache(maxsize=None)
def _sc_edge_fn(Nn, E):
    C = 80
    ZR = 125
    rows_pt = Nn // NS
    n_zc = rows_pt // ZR
    per_tile = E // NS
    n_ch = per_tile // C
    assert rows_pt % ZR == 0 and per_tile % C == 0

    def body(hm, em, ew, src, dst, agg, src_v, dst_v, w_v, g_v, e_v, z_v, acc,
             sem, sem2, sem3, sem4):
        c = lax.axis_index("c")
        s = lax.axis_index("s")
        zero = jnp.zeros((LN,), F32)

        def zrow(i, carry):
            for f in range(HH // LN):
                z_v[i, pl.ds(f * LN, LN)] = zero
            return carry

        lax.fori_loop(0, ZR, zrow, 0)

        def zcopy(j, carry):
            pltpu.sync_copy(z_v, acc.at[pl.ds(s * rows_pt + j * ZR, ZR)])
            return carry

        lax.fori_loop(0, n_zc, zcopy, 0)
        plsc.subcore_barrier()

        base = s * per_tile
        coff = c * Nn

        def chunk(j, carry):
            off = base + j * C
            a1 = pltpu.async_copy(src.at[pl.ds(off, C)], src_v, sem)
            a2 = pltpu.async_copy(dst.at[pl.ds(off, C)], dst_v, sem2)
            a3 = pltpu.async_copy(ew.at[pl.ds(off, C)], w_v, sem3)
            a4 = pltpu.async_copy(em.at[pl.ds(c * E + off, C)], e_v, sem4)
            a1.wait()
            for k in range(C // LN):
                src_v[pl.ds(k * LN, LN)] = src_v[pl.ds(k * LN, LN)] + coff
            pltpu.async_copy(hm.at[src_v], g_v, sem).wait()
            a3.wait()
            a4.wait()

            def row(i, carry2):
                wspl = plsc.load_gather(w_v, [jnp.zeros((LN,), jnp.int32) + i])
                for f in range(HH // LN):
                    g = g_v[i, pl.ds(f * LN, LN)]
                    e = e_v[i, pl.ds(f * LN, LN)]
                    g_v[i, pl.ds(f * LN, LN)] = jnp.maximum((g + e) * wspl, 0.0)
                return carry2

            lax.fori_loop(0, C, row, 0)
            a2.wait()
            pltpu.sync_copy(g_v, acc.at[dst_v], add=True)
            return carry

        lax.fori_loop(0, n_ch, chunk, 0)
        plsc.subcore_barrier()

        def wout(j, carry):
            r0 = s * rows_pt + j * ZR
            pltpu.sync_copy(acc.at[pl.ds(r0, ZR)], agg.at[pl.ds(coff + r0, ZR)])
            return carry

        lax.fori_loop(0, n_zc, wout, 0)

    return pl.kernel(
        body,
        out_type=jax.ShapeDtypeStruct((NC * Nn, HH), F32),
        mesh=_mesh(),
        compiler_params=pltpu.CompilerParams(use_tc_tiling_on_sc=False, needs_layout_passes=False),
        scratch_types=[
            pltpu.VMEM((C,), jnp.int32),
            pltpu.VMEM((C,), jnp.int32),
            pltpu.VMEM((C,), F32),
            pltpu.VMEM((C, HH), F32),
            pltpu.VMEM((C, HH), F32),
            pltpu.VMEM((ZR, HH), F32),
            pltpu.VMEM_SHARED((Nn, HH), F32),
            pltpu.SemaphoreType.DMA,
            pltpu.SemaphoreType.DMA,
            pltpu.SemaphoreType.DMA,
            pltpu.SemaphoreType.DMA,
        ])


def _tc_in_proj(x, rw, W_in, W_rw, b):
    N = x.shape[0]
    Bn = 2000

    def body(x_ref, rw_ref, wi_ref, wr_ref, b_ref, o_ref):
        o_ref[...] = (
            jnp.dot(x_ref[...], wi_ref[...], preferred_element_type=F32)
            + jnp.dot(rw_ref[...], wr_ref[...], preferred_element_type=F32)
            + b_ref[...])

    return pl.pallas_call(
        body,
        grid=(N // Bn,),
        in_specs=[
            pl.BlockSpec((Bn, H), lambda j: (j, 0)),
            pl.BlockSpec((Bn, 16), lambda j: (j, 0)),
            pl.BlockSpec((H, H), lambda j: (0, 0)),
            pl.BlockSpec((16, H), lambda j: (0, 0)),
            pl.BlockSpec((1, H), lambda j: (0, 0)),
        ],
        out_specs=pl.BlockSpec((Bn, H), lambda j: (j, 0)),
        out_shape=jax.ShapeDtypeStruct((N, H), F32))(x, rw, W_in, W_rw, b)


def _tc_halves_mm(a, W2, b2):
    M, K = a.shape
    Bm = 4000 if M % 4000 == 0 else 2000
    nj = M // Bm

    def body(a_ref, w_ref, b_ref, o_ref):
        o_ref[...] = jnp.dot(a_ref[...], w_ref[0], preferred_element_type=F32) + b_ref[0]

    return pl.pallas_call(
        body,
        grid=(2, nj),
        in_specs=[
            pl.BlockSpec((Bm, K), lambda c, j: (j, 0)),
            pl.BlockSpec((1, K, HH), lambda c, j: (c, 0, 0)),
            pl.BlockSpec((1, 1, HH), lambda c, j: (c, 0, 0)),
        ],
        out_specs=pl.BlockSpec((Bm, HH), lambda c, j: (c * nj + j, 0)),
        out_shape=jax.ShapeDtypeStruct((2 * M, HH), F32))(a, W2, b2)


def _tc_update(h, agg, Ws, Wa2, bu):
    Nn = h.shape[0]
    Bn = 2000
    nj = Nn // Bn

    def body(h_ref, a0_ref, a1_ref, ws_ref, wa0_ref, wa1_ref, b_ref, o_ref):
        acc = jnp.dot(h_ref[...], ws_ref[...], preferred_element_type=F32)
        acc = acc + jnp.dot(a0_ref[...], wa0_ref[0], preferred_element_type=F32)
        acc = acc + jnp.dot(a1_ref[...], wa1_ref[0], preferred_element_type=F32)
        o_ref[...] = jnp.maximum(acc + b_ref[...], 0.0)

    return pl.pallas_call(
        body,
        grid=(nj,),
        in_specs=[
            pl.BlockSpec((Bn, H), lambda j: (j, 0)),
            pl.BlockSpec((Bn, HH), lambda j: (j, 0)),
            pl.BlockSpec((Bn, HH), lambda j: (nj + j, 0)),
            pl.BlockSpec((H, H), lambda j: (0, 0)),
            pl.BlockSpec((1, HH, H), lambda j: (0, 0, 0)),
            pl.BlockSpec((1, HH, H), lambda j: (1, 0, 0)),
            pl.BlockSpec((1, H), lambda j: (0, 0)),
        ],
        out_specs=pl.BlockSpec((Bn, H), lambda j: (j, 0)),
        out_shape=jax.ShapeDtypeStruct((Nn, H), F32))(h, agg, agg, Ws, Wa2, Wa2, bu)


def _tc_segmean(v, ids2):
    M = v.shape[0]
    Bm = 2000
    nj = M // Bm

    def body(v_ref, id_ref, o_ref, acc, cnt):
        j = pl.program_id(0)

        @pl.when(j == 0)
        def _():
            acc[...] = jnp.zeros_like(acc)
            cnt[...] = jnp.zeros_like(cnt)

        oh = (id_ref[...] == lax.broadcasted_iota(jnp.int32, (Bm, 16), 1)).astype(F32)
        acc[...] += lax.dot_general(oh, v_ref[...], (((0,), (0,)), ((), ())),
                                    preferred_element_type=F32, precision=lax.Precision.HIGHEST)
        cnt[...] += jnp.sum(oh, axis=0)[:, None]

        @pl.when(j == nj - 1)
        def _():
            o_ref[...] = acc[...] / jnp.maximum(cnt[...], 1.0)

    return pl.pallas_call(
        body,
        grid=(nj,),
        in_specs=[
            pl.BlockSpec((Bm, H), lambda j: (j, 0)),
            pl.BlockSpec((Bm, 1), lambda j: (j, 0)),
        ],
        out_specs=pl.BlockSpec((16, H), lambda j: (0, 0)),
        out_shape=jax.ShapeDtypeStruct((16, H), F32),
        scratch_shapes=[pltpu.VMEM((16, H), F32), pltpu.VMEM((16, H), F32)])(v, ids2)


def _tc_head(emb_sub, sub_full, patch_pe, ctx_idx2, tgt_idx2, W_prw, b_prw,
             w1, b1, g1, be1, w2, b2, g2, be2, w3, b3):
    B, NTP = 16, 4

    def body(es, sf, pe, ci, ti, wpr, bpr, w1r, b1r, g1r, be1r,
             w2r, b2r, g2r, be2r, w3r, b3r, o_tgt, o_pred, o_ctx):
        ohc = (ci[...] == lax.broadcasted_iota(jnp.int32, (B, 16), 1)).astype(F32)
        emb_ctx = jnp.dot(ohc, es[...], preferred_element_type=F32, precision=lax.Precision.HIGHEST)
        oht = (ti[...] == lax.broadcasted_iota(jnp.int32, (B * NTP, 16), 1)).astype(F32)
        emb_tgt = jnp.dot(oht, sf[...], preferred_element_type=F32, precision=lax.Precision.HIGHEST)
        tpes = jnp.dot(jnp.dot(oht, pe[...], preferred_element_type=F32, precision=lax.Precision.HIGHEST), wpr[...],
                       preferred_element_type=F32) + bpr[...]
        ohr = ((lax.broadcasted_iota(jnp.int32, (B * NTP, 16), 0) // NTP)
               == lax.broadcasted_iota(jnp.int32, (B * NTP, 16), 1)).astype(F32)
        cond = jnp.dot(ohr, emb_ctx, preferred_element_type=F32, precision=lax.Precision.HIGHEST) + tpes

        def bn_relu(hh, g, bb):
            m = jnp.mean(hh, axis=0, keepdims=True)
            var = jnp.mean((hh - m) ** 2, axis=0, keepdims=True)
            return jnp.maximum((hh - m) / jnp.sqrt(var + 1e-5) * g + bb, 0.0)

        h1 = bn_relu(jnp.dot(cond, w1r[...], preferred_element_type=F32) + b1r[...],
                     g1r[...], be1r[...])
        h2 = bn_relu(jnp.dot(h1, w2r[...], preferred_element_type=F32) + b2r[...],
                     g2r[...], be2r[...])
        o_pred[...] = jnp.dot(h2, w3r[...], preferred_element_type=F32) + b3r[...]
        o_tgt[...] = emb_tgt
        o_ctx[...] = emb_ctx

    return pl.pallas_call(
        body,
        out_shape=(
            jax.ShapeDtypeStruct((B * NTP, H), F32),
            jax.ShapeDtypeStruct((B * NTP, H), F32),
            jax.ShapeDtypeStruct((B, H), F32),
        ))(emb_sub, sub_full, patch_pe, ctx_idx2, tgt_idx2, W_prw, b_prw,
           w1, b1, g1, be1, w2, b2, g2, be2, w3, b3)


def _run_mpnn(h, src, dst, ea, ew, pm, Nn, Ee):
    for l in range(3):
        Wm2 = pm["Wm"][l].reshape(H, 2, HH).transpose(1, 0, 2)
        bm2 = pm["bm"][l].reshape(2, 1, HH)
        We2 = pm["We"][l].reshape(16, 2, HH).transpose(1, 0, 2)
        hm = _tc_halves_mm(h, Wm2, bm2)
        em = _tc_halves_mm(ea, We2, jnp.zeros((2, 1, HH), F32))
        agg = _sc_edge_fn(Nn, Ee)(hm, em, ew, src, dst)
        Wa2 = pm["Wa"][l].reshape(2, HH, H)
        h = _tc_update(h, agg, pm["Ws"][l], Wa2, pm["bu"][l].reshape(1, H))
    return h


def kernel(x, rw_pos_enc, edge_attr, edge_weight, node_weight, patch_pe, y_EA,
           params, edge_index, batch, subgraphs_nodes_mapper, combined_subgraphs,
           subgraphs_edges_mapper, subgraphs_batch, call_n_patches,
           context_subgraph_idx, target_subgraph_idxs):
    p = params
    NN = x.shape[0]
    NSUB = subgraphs_nodes_mapper.shape[0]
    E = edge_index.shape[1]
    ES = combined_subgraphs.shape[1]

    b0 = (p["b_in"] + p["b_rw"]).reshape(1, H)
    x0 = _tc_in_proj(x, rw_pos_enc, p["W_in"], p["W_rw"], b0)

    gran = 8 * NC * NS * 10
    Mp = ((NSUB + gran - 1) // gran) * gran
    snm_p = jnp.concatenate([
        subgraphs_nodes_mapper.astype(jnp.int32),
        jnp.zeros((Mp - NSUB,), jnp.int32)])
    xs = _sc_gather_fn(Mp, H)(x0, snm_p)[:NSUB]

    sem_idx = subgraphs_edges_mapper.astype(jnp.int32)
    ea_in, _ = lax.optimization_barrier((edge_attr, xs))
    ea_s = _sc_gather_fn(ES, 16)(ea_in, sem_idx)
    ew16 = jnp.broadcast_to(edge_weight[:, None], (E, 16))
    ew_in, _ = lax.optimization_barrier((ew16, ea_s))
    ew_s = _sc_gather_fn(ES, 16)(ew_in, sem_idx)[:, 0]

    xc = _run_mpnn(xs, combined_subgraphs[0].astype(jnp.int32),
                   combined_subgraphs[1].astype(jnp.int32),
                   ea_s, ew_s, p["ctx"], NSUB, ES)
    sb2 = subgraphs_batch.astype(jnp.int32).reshape(NSUB, 1)
    emb_sub = _tc_segmean(xc, sb2)

    x0t, _ = lax.optimization_barrier((x0, xc))
    full_emb = _run_mpnn(x0t, edge_index[0].astype(jnp.int32),
                         edge_index[1].astype(jnp.int32),
                         edge_attr, edge_weight, p["tgt"], NN, E)
    vis_graph = _tc_segmean(full_emb, batch.astype(jnp.int32).reshape(NN, 1))
    fes = _sc_gather_fn(Mp, H)(full_emb, snm_p)[:NSUB]
    sub_full = _tc_segmean(fes, sb2)

    cnp = call_n_patches
    bi = jnp.concatenate([jnp.zeros((1,), cnp.dtype), jnp.cumsum(cnp)[:-1]])
    ctx_idx2 = (context_subgraph_idx + bi).astype(jnp.int32).reshape(16, 1)
    tgt_idx2 = (target_subgraph_idxs + bi[:, None]).reshape(-1).astype(jnp.int32).reshape(64, 1)

    emb_tgt64, pred64, emb_ctx = _tc_head(
        emb_sub, sub_full, patch_pe, ctx_idx2, tgt_idx2,
        p["W_prw"], p["b_prw"].reshape(1, H),
        p["pW1"], p["pb1"].reshape(1, H), p["pg1"].reshape(1, H), p["pbe1"].reshape(1, H),
        p["pW2"], p["pb2"].reshape(1, H), p["pg2"].reshape(1, H), p["pbe2"].reshape(1, H),
        p["pW3"], p["pb3"].reshape(1, H))

    emb_tgt = emb_tgt64.reshape(16, 4, H)
    pred = pred64.reshape(16, 4, H)
    vis_tgt = emb_tgt[:, 0, :]
    return (emb_tgt, pred, emb_ctx, vis_tgt, vis_graph)


# speedup vs baseline: 1.4248x; 1.1593x over previous
import functools

import jax
import jax.numpy as jnp
from jax import lax
from jax.experimental import pallas as pl
from jax.experimental.pallas import tpu as pltpu
from jax.experimental.pallas import tpu_sc as plsc

NC = 2
NS = 16
LN = 16
H = 128
HH = 64
F32 = jnp.float32


def _mesh():
    return plsc.VectorSubcoreMesh(
        core_axis_name="c", subcore_axis_name="s", num_cores=NC, num_subcores=NS)


@functools.lru_cache(maxsize=None)
def _sc_gather_fn(M, D):
    C = 80
    NW = NC * NS
    per_w = M // NW
    n_ch = per_w // C
    assert per_w % C == 0 and per_w * NW == M

    def body(table, idx, out, idx_v, rows_v, sem):
        wid = lax.axis_index("s") * NC + lax.axis_index("c")
        base = wid * per_w

        def chunk(j, carry):
            off = base + j * C
            pltpu.sync_copy(idx.at[pl.ds(off, C)], idx_v)
            pltpu.async_copy(table.at[idx_v], rows_v, sem).wait()
            pltpu.sync_copy(rows_v, out.at[pl.ds(off, C)])
            return carry

        lax.fori_loop(0, n_ch, chunk, 0)

    return pl.kernel(
        body,
        out_type=jax.ShapeDtypeStruct((M, D), F32),
        mesh=_mesh(),
        compiler_params=pltpu.CompilerParams(use_tc_tiling_on_sc=False, needs_layout_passes=False),
        scratch_types=[
            pltpu.VMEM((C,), jnp.int32),
            pltpu.VMEM((C, D), F32),
            pltpu.SemaphoreType.DMA,
        ])


@functools.lru_cache(maxsize=None)
def _sc_edge_fn(Nn, E):
    C = 80
    ZR = 125
    rows_pt = Nn // NS
    n_zc = rows_pt // ZR
    per_tile = E // NS
    n_ch = per_tile // C
    assert rows_pt % ZR == 0 and per_tile % C == 0

    def body(hm, em, ew, src, dst, agg, src_v, dst_v, w_v, g_v, e_v,
             src_vb, dst_vb, w_vb, g_vb, e_vb, z_v, acc,
             sem, sem2, sem3, sem4, semb, sem2b, sem3b, sem4b):
        c = lax.axis_index("c")
        s = lax.axis_index("s")
        zero = jnp.zeros((LN,), F32)

        def zrow(i, carry):
            for f in range(HH // LN):
                z_v[i, pl.ds(f * LN, LN)] = zero
            return carry

        lax.fori_loop(0, ZR, zrow, 0)

        def zcopy(j, carry):
            pltpu.sync_copy(z_v, acc.at[pl.ds(s * rows_pt + j * ZR, ZR)])
            return carry

        lax.fori_loop(0, n_zc, zcopy, 0)
        plsc.subcore_barrier()

        base = s * per_tile
        coff = c * Nn
        bufs = ((src_v, dst_v, w_v, g_v, e_v, sem, sem2, sem3, sem4),
                (src_vb, dst_vb, w_vb, g_vb, e_vb, semb, sem2b, sem3b, sem4b))

        def issue(b, j):
            sv, dv, wv, gv, ev, s1, s2, s3, s4 = bufs[b]
            off = base + j * C
            a1 = pltpu.async_copy(src.at[pl.ds(off, C)], sv, s1)
            pltpu.async_copy(dst.at[pl.ds(off, C)], dv, s2)
            pltpu.async_copy(ew.at[pl.ds(off, C)], wv, s3)
            pltpu.async_copy(em.at[pl.ds(c * E + off, C)], ev, s4)
            a1.wait()
            for k in range(C // LN):
                sv[pl.ds(k * LN, LN)] = sv[pl.ds(k * LN, LN)] + coff
            return pltpu.async_copy(hm.at[sv], gv, s1)

        def consume(b, gdesc):
            sv, dv, wv, gv, ev, s1, s2, s3, s4 = bufs[b]
            gdesc.wait()
            pltpu.make_async_copy(ew.at[pl.ds(base, C)], wv, s3).wait()
            pltpu.make_async_copy(em.at[pl.ds(base, C)], ev, s4).wait()

            def row(i, carry2):
                wspl = plsc.load_gather(wv, [jnp.zeros((LN,), jnp.int32) + i])
                for f in range(HH // LN):
                    g = gv[i, pl.ds(f * LN, LN)]
                    e = ev[i, pl.ds(f * LN, LN)]
                    gv[i, pl.ds(f * LN, LN)] = jnp.maximum((g + e) * wspl, 0.0)
                return carry2

            lax.fori_loop(0, C, row, 0)
            pltpu.make_async_copy(dst.at[pl.ds(base, C)], dv, s2).wait()
            pltpu.sync_copy(gv, acc.at[dv], add=True)

        def pair(p, carry):
            j0 = p * 2
            g0 = issue(0, j0)
            g1 = issue(1, j0 + 1)
            consume(0, g0)
            consume(1, g1)
            return carry

        lax.fori_loop(0, n_ch // 2, pair, 0)
        plsc.subcore_barrier()

        def wout(j, carry):
            r0 = s * rows_pt + j * ZR
            pltpu.sync_copy(acc.at[pl.ds(r0, ZR)], agg.at[pl.ds(coff + r0, ZR)])
            return carry

        lax.fori_loop(0, n_zc, wout, 0)

    return pl.kernel(
        body,
        out_type=jax.ShapeDtypeStruct((NC * Nn, HH), F32),
        mesh=_mesh(),
        compiler_params=pltpu.CompilerParams(use_tc_tiling_on_sc=False, needs_layout_passes=False),
        scratch_types=[
            pltpu.VMEM((C,), jnp.int32),
            pltpu.VMEM((C,), jnp.int32),
            pltpu.VMEM((C,), F32),
            pltpu.VMEM((C, HH), F32),
            pltpu.VMEM((C, HH), F32),
            pltpu.VMEM((C,), jnp.int32),
            pltpu.VMEM((C,), jnp.int32),
            pltpu.VMEM((C,), F32),
            pltpu.VMEM((C, HH), F32),
            pltpu.VMEM((C, HH), F32),
            pltpu.VMEM((ZR, HH), F32),
            pltpu.VMEM_SHARED((Nn, HH), F32),
            pltpu.SemaphoreType.DMA,
            pltpu.SemaphoreType.DMA,
            pltpu.SemaphoreType.DMA,
            pltpu.SemaphoreType.DMA,
            pltpu.SemaphoreType.DMA,
            pltpu.SemaphoreType.DMA,
            pltpu.SemaphoreType.DMA,
            pltpu.SemaphoreType.DMA,
        ])


def _tc_in_proj(x, rw, W_in, W_rw, b):
    N = x.shape[0]
    Bn = 2000

    def body(x_ref, rw_ref, wi_ref, wr_ref, b_ref, o_ref):
        o_ref[...] = (
            jnp.dot(x_ref[...], wi_ref[...], preferred_element_type=F32)
            + jnp.dot(rw_ref[...], wr_ref[...], preferred_element_type=F32)
            + b_ref[...])

    return pl.pallas_call(
        body,
        grid=(N // Bn,),
        in_specs=[
            pl.BlockSpec((Bn, H), lambda j: (j, 0)),
            pl.BlockSpec((Bn, 16), lambda j: (j, 0)),
            pl.BlockSpec((H, H), lambda j: (0, 0)),
            pl.BlockSpec((16, H), lambda j: (0, 0)),
            pl.BlockSpec((1, H), lambda j: (0, 0)),
        ],
        out_specs=pl.BlockSpec((Bn, H), lambda j: (j, 0)),
        out_shape=jax.ShapeDtypeStruct((N, H), F32))(x, rw, W_in, W_rw, b)


def _tc_halves_mm(a, W2, b2):
    M, K = a.shape
    Bm = 4000 if M % 4000 == 0 else 2000
    nj = M // Bm

    def body(a_ref, w_ref, b_ref, o_ref):
        o_ref[...] = jnp.dot(a_ref[...], w_ref[0], preferred_element_type=F32) + b_ref[0]

    return pl.pallas_call(
        body,
        grid=(2, nj),
        in_specs=[
            pl.BlockSpec((Bm, K), lambda c, j: (j, 0)),
            pl.BlockSpec((1, K, HH), lambda c, j: (c, 0, 0)),
            pl.BlockSpec((1, 1, HH), lambda c, j: (c, 0, 0)),
        ],
        out_specs=pl.BlockSpec((Bm, HH), lambda c, j: (c * nj + j, 0)),
        out_shape=jax.ShapeDtypeStruct((2 * M, HH), F32))(a, W2, b2)


def _tc_update(h, agg, Ws, Wa2, bu):
    Nn = h.shape[0]
    Bn = 2000
    nj = Nn // Bn

    def body(h_ref, a0_ref, a1_ref, ws_ref, wa0_ref, wa1_ref, b_ref, o_ref):
        acc = jnp.dot(h_ref[...], ws_ref[...], preferred_element_type=F32)
        acc = acc + jnp.dot(a0_ref[...], wa0_ref[0], preferred_element_type=F32)
        acc = acc + jnp.dot(a1_ref[...], wa1_ref[0], preferred_element_type=F32)
        o_ref[...] = jnp.maximum(acc + b_ref[...], 0.0)

    return pl.pallas_call(
        body,
        grid=(nj,),
        in_specs=[
            pl.BlockSpec((Bn, H), lambda j: (j, 0)),
            pl.BlockSpec((Bn, HH), lambda j: (j, 0)),
            pl.BlockSpec((Bn, HH), lambda j: (nj + j, 0)),
            pl.BlockSpec((H, H), lambda j: (0, 0)),
            pl.BlockSpec((1, HH, H), lambda j: (0, 0, 0)),
            pl.BlockSpec((1, HH, H), lambda j: (1, 0, 0)),
            pl.BlockSpec((1, H), lambda j: (0, 0)),
        ],
        out_specs=pl.BlockSpec((Bn, H), lambda j: (j, 0)),
        out_shape=jax.ShapeDtypeStruct((Nn, H), F32))(h, agg, agg, Ws, Wa2, Wa2, bu)


def _tc_segmean(v, ids2):
    M = v.shape[0]
    Bm = 2000
    nj = M // Bm

    def body(v_ref, id_ref, o_ref, acc, cnt):
        j = pl.program_id(0)

        @pl.when(j == 0)
        def _():
            acc[...] = jnp.zeros_like(acc)
            cnt[...] = jnp.zeros_like(cnt)

        oh = (id_ref[...] == lax.broadcasted_iota(jnp.int32, (Bm, 16), 1)).astype(F32)
        acc[...] += lax.dot_general(oh, v_ref[...], (((0,), (0,)), ((), ())),
                                    preferred_element_type=F32, precision=lax.Precision.HIGHEST)
        cnt[...] += jnp.sum(oh, axis=0)[:, None]

        @pl.when(j == nj - 1)
        def _():
            o_ref[...] = acc[...] / jnp.maximum(cnt[...], 1.0)

    return pl.pallas_call(
        body,
        grid=(nj,),
        in_specs=[
            pl.BlockSpec((Bm, H), lambda j: (j, 0)),
            pl.BlockSpec((Bm, 1), lambda j: (j, 0)),
        ],
        out_specs=pl.BlockSpec((16, H), lambda j: (0, 0)),
        out_shape=jax.ShapeDtypeStruct((16, H), F32),
        scratch_shapes=[pltpu.VMEM((16, H), F32), pltpu.VMEM((16, H), F32)])(v, ids2)


def _tc_head(emb_sub, sub_full, patch_pe, ctx_idx2, tgt_idx2, W_prw, b_prw,
             w1, b1, g1, be1, w2, b2, g2, be2, w3, b3):
    B, NTP = 16, 4

    def body(es, sf, pe, ci, ti, wpr, bpr, w1r, b1r, g1r, be1r,
             w2r, b2r, g2r, be2r, w3r, b3r, o_tgt, o_pred, o_ctx):
        ohc = (ci[...] == lax.broadcasted_iota(jnp.int32, (B, 16), 1)).astype(F32)
        emb_ctx = jnp.dot(ohc, es[...], preferred_element_type=F32, precision=lax.Precision.HIGHEST)
        oht = (ti[...] == lax.broadcasted_iota(jnp.int32, (B * NTP, 16), 1)).astype(F32)
        emb_tgt = jnp.dot(oht, sf[...], preferred_element_type=F32, precision=lax.Precision.HIGHEST)
        tpes = jnp.dot(jnp.dot(oht, pe[...], preferred_element_type=F32, precision=lax.Precision.HIGHEST), wpr[...],
                       preferred_element_type=F32) + bpr[...]
        ohr = ((lax.broadcasted_iota(jnp.int32, (B * NTP, 16), 0) // NTP)
               == lax.broadcasted_iota(jnp.int32, (B * NTP, 16), 1)).astype(F32)
        cond = jnp.dot(ohr, emb_ctx, preferred_element_type=F32, precision=lax.Precision.HIGHEST) + tpes

        def bn_relu(hh, g, bb):
            m = jnp.mean(hh, axis=0, keepdims=True)
            var = jnp.mean((hh - m) ** 2, axis=0, keepdims=True)
            return jnp.maximum((hh - m) / jnp.sqrt(var + 1e-5) * g + bb, 0.0)

        h1 = bn_relu(jnp.dot(cond, w1r[...], preferred_element_type=F32) + b1r[...],
                     g1r[...], be1r[...])
        h2 = bn_relu(jnp.dot(h1, w2r[...], preferred_element_type=F32) + b2r[...],
                     g2r[...], be2r[...])
        o_pred[...] = jnp.dot(h2, w3r[...], preferred_element_type=F32) + b3r[...]
        o_tgt[...] = emb_tgt
        o_ctx[...] = emb_ctx

    return pl.pallas_call(
        body,
        out_shape=(
            jax.ShapeDtypeStruct((B * NTP, H), F32),
            jax.ShapeDtypeStruct((B * NTP, H), F32),
            jax.ShapeDtypeStruct((B, H), F32),
        ))(emb_sub, sub_full, patch_pe, ctx_idx2, tgt_idx2, W_prw, b_prw,
           w1, b1, g1, be1, w2, b2, g2, be2, w3, b3)


def _run_mpnn(h, src, dst, ea, ew, pm, Nn, Ee):
    for l in range(3):
        Wm2 = pm["Wm"][l].reshape(H, 2, HH).transpose(1, 0, 2)
        bm2 = pm["bm"][l].reshape(2, 1, HH)
        We2 = pm["We"][l].reshape(16, 2, HH).transpose(1, 0, 2)
        hm = _tc_halves_mm(h, Wm2, bm2)
        em = _tc_halves_mm(ea, We2, jnp.zeros((2, 1, HH), F32))
        agg = _sc_edge_fn(Nn, Ee)(hm, em, ew, src, dst)
        Wa2 = pm["Wa"][l].reshape(2, HH, H)
        h = _tc_update(h, agg, pm["Ws"][l], Wa2, pm["bu"][l].reshape(1, H))
    return h


def kernel(x, rw_pos_enc, edge_attr, edge_weight, node_weight, patch_pe, y_EA,
           params, edge_index, batch, subgraphs_nodes_mapper, combined_subgraphs,
           subgraphs_edges_mapper, subgraphs_batch, call_n_patches,
           context_subgraph_idx, target_subgraph_idxs):
    p = params
    NN = x.shape[0]
    NSUB = subgraphs_nodes_mapper.shape[0]
    E = edge_index.shape[1]
    ES = combined_subgraphs.shape[1]

    b0 = (p["b_in"] + p["b_rw"]).reshape(1, H)
    x0 = _tc_in_proj(x, rw_pos_enc, p["W_in"], p["W_rw"], b0)

    gran = 8 * NC * NS * 10
    Mp = ((NSUB + gran - 1) // gran) * gran
    snm_p = jnp.concatenate([
        subgraphs_nodes_mapper.astype(jnp.int32),
        jnp.zeros((Mp - NSUB,), jnp.int32)])
    xs = _sc_gather_fn(Mp, H)(x0, snm_p)[:NSUB]

    sem_idx = subgraphs_edges_mapper.astype(jnp.int32)
    ea_in, _ = lax.optimization_barrier((edge_attr, xs))
    ea_s = _sc_gather_fn(ES, 16)(ea_in, sem_idx)
    ew16 = jnp.broadcast_to(edge_weight[:, None], (E, 16))
    ew_in, _ = lax.optimization_barrier((ew16, ea_s))
    ew_s = _sc_gather_fn(ES, 16)(ew_in, sem_idx)[:, 0]

    xc = _run_mpnn(xs, combined_subgraphs[0].astype(jnp.int32),
                   combined_subgraphs[1].astype(jnp.int32),
                   ea_s, ew_s, p["ctx"], NSUB, ES)
    sb2 = subgraphs_batch.astype(jnp.int32).reshape(NSUB, 1)
    emb_sub = _tc_segmean(xc, sb2)

    x0t, _ = lax.optimization_barrier((x0, xc))
    full_emb = _run_mpnn(x0t, edge_index[0].astype(jnp.int32),
                         edge_index[1].astype(jnp.int32),
                         edge_attr, edge_weight, p["tgt"], NN, E)
    vis_graph = _tc_segmean(full_emb, batch.astype(jnp.int32).reshape(NN, 1))
    fes = _sc_gather_fn(Mp, H)(full_emb, snm_p)[:NSUB]
    sub_full = _tc_segmean(fes, sb2)

    cnp = call_n_patches
    bi = jnp.concatenate([jnp.zeros((1,), cnp.dtype), jnp.cumsum(cnp)[:-1]])
    ctx_idx2 = (context_subgraph_idx + bi).astype(jnp.int32).reshape(16, 1)
    tgt_idx2 = (target_subgraph_idxs + bi[:, None]).reshape(-1).astype(jnp.int32).reshape(64, 1)

    emb_tgt64, pred64, emb_ctx = _tc_head(
        emb_sub, sub_full, patch_pe, ctx_idx2, tgt_idx2,
        p["W_prw"], p["b_prw"].reshape(1, H),
        p["pW1"], p["pb1"].reshape(1, H), p["pg1"].reshape(1, H), p["pbe1"].reshape(1, H),
        p["pW2"], p["pb2"].reshape(1, H), p["pg2"].reshape(1, H), p["pbe2"].reshape(1, H),
        p["pW3"], p["pb3"].reshape(1, H))

    emb_tgt = emb_tgt64.reshape(16, 4, H)
    pred = pred64.reshape(16, 4, H)
    vis_tgt = emb_tgt[:, 0, :]
    return (emb_tgt, pred, emb_ctx, vis_tgt, vis_graph)
